# baseline (speedup 1.0000x reference)
import functools

import jax
import jax.numpy as jnp
from jax import lax
from jax.experimental import pallas as pl
from jax.experimental.pallas import tpu as pltpu
from jax.experimental.pallas import tpu_sc as plsc

N = 10000
E = 320000
D = 128

NC = 2
NS = 16
NW = NC * NS
EPW = E // NW
CH = 128
NCHUNK = 78
CT = EPW - NCHUNK * CH
NBUF = 2
NIBUF = 4
NMAIN = 76
N_PAD = 10240
RPS = N_PAD // NS
BR = 2000


def _sc_segment_sum_body(x_hbm, sd_hbm, tail_hbm, zeros_hbm, out_hbm,
                         buf0, buf1, ib0, ib1, ib2, ib3, tib, tbuf,
                         gsem0, gsem1, isem0, isem1, isem2, isem3,
                         tsem, zsem, acc):
    c = lax.axis_index("c")
    s = lax.axis_index("s")
    wid = s * NC + c
    bufs = (buf0, buf1)
    gsems = (gsem0, gsem1)
    ibufs = (ib0, ib1, ib2, ib3)
    isems = (isem0, isem1, isem2, isem3)

    def fire_idx(b, k):
        pltpu.async_copy(sd_hbm.at[wid, k], ibufs[b], isems[b])

    def wait_idx(b):
        pltpu.make_async_copy(sd_hbm.at[wid, 0], ibufs[b], isems[b]).wait()

    def fire_gather(b2, b4):
        pltpu.async_copy(x_hbm.at[ibufs[b4].at[0]], bufs[b2], gsems[b2])

    def wait_gather(b2, b4):
        pltpu.make_async_copy(x_hbm.at[ibufs[b4].at[0]], bufs[b2],
                              gsems[b2]).wait()

    def scatter(b2, b4):
        pltpu.sync_copy(bufs[b2], acc.at[ibufs[b4].at[1]], add=True)

    pltpu.async_copy(zeros_hbm, acc.at[pl.ds(s * RPS, RPS)], zsem)

    for b in range(NIBUF):
        fire_idx(b, b)
    pltpu.sync_copy(tail_hbm.at[wid], tib)
    pltpu.async_copy(x_hbm.at[tib.at[0]], tbuf, tsem)
    for b in range(NBUF):
        wait_idx(b)
        fire_gather(b, b)

    pltpu.make_async_copy(zeros_hbm, acc.at[pl.ds(s * RPS, RPS)],
                          zsem).wait()
    plsc.subcore_barrier()

    @pl.loop(0, NMAIN, step=NIBUF)
    def _chunk(j):
        for b in range(NIBUF):
            k = j + b
            b2 = b % NBUF
            wait_gather(b2, b)
            scatter(b2, b)

            @pl.when(k + NIBUF < NCHUNK)
            def _prefetch_idx():
                fire_idx(b, k + NIBUF)

            @pl.when(k + NBUF < NCHUNK)
            def _refill():
                bn = (b + NBUF) % NIBUF
                wait_idx(bn)
                fire_gather(b2, bn)

    for k in (NMAIN, NMAIN + 1):
        b = k % NIBUF
        b2 = k % NBUF
        wait_gather(b2, b)
        scatter(b2, b)
    pltpu.make_async_copy(x_hbm.at[tib.at[0]], tbuf, tsem).wait()
    pltpu.sync_copy(tbuf, acc.at[tib.at[1]], add=True)

    plsc.subcore_barrier()
    pltpu.sync_copy(acc.at[pl.ds(s * RPS, RPS)],
                    out_hbm.at[c, pl.ds(s * RPS, RPS)])


@functools.lru_cache(maxsize=None)
def _sc_segment_sum():
    return pl.kernel(
        _sc_segment_sum_body,
        out_type=jax.ShapeDtypeStruct((NC, N_PAD, D), jnp.float32),
        mesh=plsc.VectorSubcoreMesh(core_axis_name="c", subcore_axis_name="s",
                                    num_cores=NC, num_subcores=NS),
        scratch_types=[pltpu.VMEM((CH, D), jnp.float32) for _ in range(NBUF)]
        + [pltpu.VMEM((2, CH), jnp.int32) for _ in range(NIBUF)]
        + [pltpu.VMEM((2, CT), jnp.int32), pltpu.VMEM((CT, D), jnp.float32)]
        + [pltpu.SemaphoreType.DMA for _ in range(NBUF + NIBUF + 2)]
        + [pltpu.VMEM_SHARED((N_PAD, D), jnp.float32)],
    )


def _tc_skip_body(x_ref, w2t_ref, b2_ref, y_ref):
    y_ref[...] = (
        jnp.dot(x_ref[...], w2t_ref[...], preferred_element_type=jnp.float32)
        + b2_ref[...]
    )


def _tc_combine_body(y_ref, p_ref, w1t_ref, o_ref):
    f1 = p_ref[0] + p_ref[1]
    o_ref[...] = y_ref[...] + jnp.dot(
        f1, w1t_ref[...], preferred_element_type=jnp.float32)


def kernel(x, edge_index, W1, W2, b2):
    srcr = edge_index[0].reshape(NW, EPW)
    dstr = edge_index[1].reshape(NW, EPW)
    main = NCHUNK * CH
    sd = jnp.stack([srcr[:, :main].reshape(NW, NCHUNK, CH),
                    dstr[:, :main].reshape(NW, NCHUNK, CH)], axis=2)
    tail = jnp.stack([srcr[:, main:], dstr[:, main:]], axis=1)
    zeros = jnp.zeros((RPS, D), dtype=jnp.float32)

    partials = _sc_segment_sum()(x, sd, tail, zeros)

    y = pl.pallas_call(
        _tc_skip_body,
        grid=(N // BR,),
        in_specs=[
            pl.BlockSpec((BR, D), lambda i: (i, 0)),
            pl.BlockSpec((D, D), lambda i: (0, 0)),
            pl.BlockSpec((1, D), lambda i: (0, 0)),
        ],
        out_specs=pl.BlockSpec((BR, D), lambda i: (i, 0)),
        out_shape=jax.ShapeDtypeStruct((N, D), jnp.float32),
    )(x, W2.T, b2.reshape(1, D))

    out = pl.pallas_call(
        _tc_combine_body,
        grid=(N // BR,),
        in_specs=[
            pl.BlockSpec((BR, D), lambda i: (i, 0)),
            pl.BlockSpec((NC, BR, D), lambda i: (0, i, 0)),
            pl.BlockSpec((D, D), lambda i: (0, 0)),
        ],
        out_specs=pl.BlockSpec((BR, D), lambda i: (i, 0)),
        out_shape=jax.ShapeDtypeStruct((N, D), jnp.float32),
    )(y, partials, W1.T)
    return out


# speedup vs baseline: 1.0792x; 1.0792x over previous
import functools

import jax
import jax.numpy as jnp
from jax import lax
from jax.experimental import pallas as pl
from jax.experimental.pallas import tpu as pltpu
from jax.experimental.pallas import tpu_sc as plsc

N = 10000
E = 320000
D = 128

NC = 2
NS = 16
NW = NC * NS
EPW = E // NW
CH = 128
NCHUNK = 78
CT = EPW - NCHUNK * CH
NBUF = 2
NIBUF = 4
NMAIN = 76
N_PAD = 10240
RPS = N_PAD // NS
BR = 2000


def _sc_segment_sum_body(x_hbm, src_hbm, dst_hbm, zeros_hbm, out_hbm,
                         buf0, buf1, ib0, ib1, ib2, ib3, tib, tbuf,
                         gsem0, gsem1, isem0, isem1, isem2, isem3,
                         tsem, zsem, acc):
    c = lax.axis_index("c")
    s = lax.axis_index("s")
    wid = s * NC + c
    bufs = (buf0, buf1)
    gsems = (gsem0, gsem1)
    ibufs = (ib0, ib1, ib2, ib3)
    isems = (isem0, isem1, isem2, isem3)

    base = wid * EPW

    def fire_idx(b, k):
        pltpu.async_copy(src_hbm.at[pl.ds(base + k * CH, CH)],
                         ibufs[b].at[0], isems[b])
        pltpu.async_copy(dst_hbm.at[pl.ds(base + k * CH, CH)],
                         ibufs[b].at[1], isems[b])

    def wait_idx(b):
        pltpu.make_async_copy(src_hbm.at[pl.ds(0, CH)],
                              ibufs[b].at[0], isems[b]).wait()
        pltpu.make_async_copy(src_hbm.at[pl.ds(0, CH)],
                              ibufs[b].at[1], isems[b]).wait()

    def fire_gather(b2, b4):
        pltpu.async_copy(x_hbm.at[ibufs[b4].at[0]], bufs[b2], gsems[b2])

    def wait_gather(b2, b4):
        pltpu.make_async_copy(x_hbm.at[ibufs[b4].at[0]], bufs[b2],
                              gsems[b2]).wait()

    def scatter(b2, b4):
        pltpu.sync_copy(bufs[b2], acc.at[ibufs[b4].at[1]], add=True)

    pltpu.async_copy(zeros_hbm, acc.at[pl.ds(s * RPS, RPS)], zsem)

    for b in range(NIBUF):
        fire_idx(b, b)
    pltpu.sync_copy(src_hbm.at[pl.ds(base + NCHUNK * CH, CT)], tib.at[0])
    pltpu.sync_copy(dst_hbm.at[pl.ds(base + NCHUNK * CH, CT)], tib.at[1])
    pltpu.async_copy(x_hbm.at[tib.at[0]], tbuf, tsem)
    for b in range(NBUF):
        wait_idx(b)
        fire_gather(b, b)

    pltpu.make_async_copy(zeros_hbm, acc.at[pl.ds(s * RPS, RPS)],
                          zsem).wait()
    plsc.subcore_barrier()

    @pl.loop(0, NMAIN, step=NIBUF)
    def _chunk(j):
        for b in range(NIBUF):
            k = j + b
            b2 = b % NBUF
            wait_gather(b2, b)
            scatter(b2, b)

            @pl.when(k + NIBUF < NCHUNK)
            def _prefetch_idx():
                fire_idx(b, k + NIBUF)

            @pl.when(k + NBUF < NCHUNK)
            def _refill():
                bn = (b + NBUF) % NIBUF
                wait_idx(bn)
                fire_gather(b2, bn)

    for k in (NMAIN, NMAIN + 1):
        b = k % NIBUF
        b2 = k % NBUF
        wait_gather(b2, b)
        scatter(b2, b)
    pltpu.make_async_copy(x_hbm.at[tib.at[0]], tbuf, tsem).wait()
    pltpu.sync_copy(tbuf, acc.at[tib.at[1]], add=True)

    plsc.subcore_barrier()
    pltpu.sync_copy(acc.at[pl.ds(s * RPS, RPS)],
                    out_hbm.at[c, pl.ds(s * RPS, RPS)])


@functools.lru_cache(maxsize=None)
def _sc_segment_sum():
    return pl.kernel(
        _sc_segment_sum_body,
        out_type=jax.ShapeDtypeStruct((NC, N_PAD, D), jnp.float32),
        mesh=plsc.VectorSubcoreMesh(core_axis_name="c", subcore_axis_name="s",
                                    num_cores=NC, num_subcores=NS),
        scratch_types=[pltpu.VMEM((CH, D), jnp.float32) for _ in range(NBUF)]
        + [pltpu.VMEM((2, CH), jnp.int32) for _ in range(NIBUF)]
        + [pltpu.VMEM((2, CT), jnp.int32), pltpu.VMEM((CT, D), jnp.float32)]
        + [pltpu.SemaphoreType.DMA for _ in range(NBUF + NIBUF + 2)]
        + [pltpu.VMEM_SHARED((N_PAD, D), jnp.float32)],
    )


def _tc_combine_body(x_ref, p_ref, w1_ref, w2_ref, b2_ref, o_ref):
    f1 = p_ref[0] + p_ref[1]
    dn = (((1,), (1,)), ((), ()))
    o_ref[...] = (
        lax.dot_general(x_ref[...], w2_ref[...], dn,
                        preferred_element_type=jnp.float32)
        + b2_ref[...]
        + lax.dot_general(f1, w1_ref[...], dn,
                          preferred_element_type=jnp.float32)
    )


def kernel(x, edge_index, W1, W2, b2):
    src = edge_index[0]
    dst = edge_index[1]
    zeros = jnp.zeros((RPS, D), dtype=jnp.float32)

    partials = _sc_segment_sum()(x, src, dst, zeros)

    out = pl.pallas_call(
        _tc_combine_body,
        grid=(N // BR,),
        in_specs=[
            pl.BlockSpec((BR, D), lambda i: (i, 0)),
            pl.BlockSpec((NC, BR, D), lambda i: (0, i, 0)),
            pl.BlockSpec((D, D), lambda i: (0, 0)),
            pl.BlockSpec((D, D), lambda i: (0, 0)),
            pl.BlockSpec((1, D), lambda i: (0, 0)),
        ],
        out_specs=pl.BlockSpec((BR, D), lambda i: (i, 0)),
        out_shape=jax.ShapeDtypeStruct((N, D), jnp.float32),
    )(x, partials, W1, W2, b2.reshape(1, D))
    return out


# speedup vs baseline: 1.1793x; 1.0927x over previous
import functools

import jax
import jax.numpy as jnp
from jax import lax
from jax.experimental import pallas as pl
from jax.experimental.pallas import tpu as pltpu
from jax.experimental.pallas import tpu_sc as plsc

N = 10000
E = 320000
D = 128

NC = 2
NS = 16
NW = NC * NS
CH = 128
NCHUNKS = E // CH
NCHUNK = NCHUNKS // NW
NEXTRA = NCHUNKS - NCHUNK * NW
NBUF = 2
NIBUF = 4
NMAIN = 76
N_PAD = 10240
RPS = N_PAD // NS
BR = 2000


def _sc_segment_sum_body(x_hbm, ei_hbm, zeros_hbm, out_hbm,
                         buf0, buf1, ib0, ib1, ib2, ib3,
                         gsem0, gsem1, isem0, isem1, isem2, isem3,
                         zsem, acc):
    c = lax.axis_index("c")
    s = lax.axis_index("s")
    wid = s * NC + c
    bufs = (buf0, buf1)
    gsems = (gsem0, gsem1)
    ibufs = (ib0, ib1, ib2, ib3)
    isems = (isem0, isem1, isem2, isem3)

    def fire_idx(b, k):
        off = (wid + NW * k) * CH
        pltpu.async_copy(ei_hbm.at[pl.ds(0, 2), pl.ds(off, CH)],
                         ibufs[b], isems[b])

    def wait_idx(b):
        pltpu.make_async_copy(ei_hbm.at[pl.ds(0, 2), pl.ds(0, CH)],
                              ibufs[b], isems[b]).wait()

    def fire_gather(b2, b4):
        pltpu.async_copy(x_hbm.at[ibufs[b4].at[0]], bufs[b2], gsems[b2])

    def wait_gather(b2, b4):
        pltpu.make_async_copy(x_hbm.at[ibufs[b4].at[0]], bufs[b2],
                              gsems[b2]).wait()

    def scatter(b2, b4):
        pltpu.sync_copy(bufs[b2], acc.at[ibufs[b4].at[1]], add=True)

    pltpu.async_copy(zeros_hbm, acc.at[pl.ds(s * RPS, RPS)], zsem)

    for b in range(NIBUF):
        fire_idx(b, b)
    for b in range(NBUF):
        wait_idx(b)
        fire_gather(b, b)

    pltpu.make_async_copy(zeros_hbm, acc.at[pl.ds(s * RPS, RPS)],
                          zsem).wait()
    plsc.subcore_barrier()

    @pl.loop(0, NMAIN, step=NIBUF)
    def _chunk(j):
        for b in range(NIBUF):
            k = j + b
            b2 = b % NBUF
            wait_gather(b2, b)
            scatter(b2, b)

            @pl.when(k + NIBUF < NCHUNK)
            def _prefetch_idx():
                fire_idx(b, k + NIBUF)

            @pl.when(k + NBUF < NCHUNK)
            def _refill():
                bn = (b + NBUF) % NIBUF
                wait_idx(bn)
                fire_gather(b2, bn)

    for k in (NMAIN, NMAIN + 1):
        b = k % NIBUF
        b2 = k % NBUF
        wait_gather(b2, b)
        scatter(b2, b)

    @pl.when(wid < NEXTRA)
    def _extra():
        off = (NCHUNK * NW + wid) * CH
        pltpu.sync_copy(ei_hbm.at[pl.ds(0, 2), pl.ds(off, CH)], ib0)
        pltpu.async_copy(x_hbm.at[ib0.at[0]], buf0, gsem0).wait()
        pltpu.sync_copy(buf0, acc.at[ib0.at[1]], add=True)

    plsc.subcore_barrier()
    pltpu.sync_copy(acc.at[pl.ds(s * RPS, RPS)],
                    out_hbm.at[c, pl.ds(s * RPS, RPS)])


@functools.lru_cache(maxsize=None)
def _sc_segment_sum():
    return pl.kernel(
        _sc_segment_sum_body,
        out_type=jax.ShapeDtypeStruct((NC, N_PAD, D), jnp.float32),
        mesh=plsc.VectorSubcoreMesh(core_axis_name="c", subcore_axis_name="s",
                                    num_cores=NC, num_subcores=NS),
        scratch_types=[pltpu.VMEM((CH, D), jnp.float32) for _ in range(NBUF)]
        + [pltpu.VMEM((2, CH), jnp.int32) for _ in range(NIBUF)]
        + [pltpu.SemaphoreType.DMA for _ in range(NBUF + NIBUF + 1)]
        + [pltpu.VMEM_SHARED((N_PAD, D), jnp.float32)],
    )


def _tc_combine_body(x_ref, p_ref, w1_ref, w2_ref, b2_ref, o_ref):
    f1 = p_ref[0] + p_ref[1]
    dn = (((1,), (1,)), ((), ()))
    o_ref[...] = (
        lax.dot_general(x_ref[...], w2_ref[...], dn,
                        preferred_element_type=jnp.float32)
        + b2_ref[...]
        + lax.dot_general(f1, w1_ref[...], dn,
                          preferred_element_type=jnp.float32)
    )


def kernel(x, edge_index, W1, W2, b2):
    zeros = jnp.zeros((RPS, D), dtype=jnp.float32)

    partials = _sc_segment_sum()(x, edge_index, zeros)

    out = pl.pallas_call(
        _tc_combine_body,
        grid=(N // BR,),
        in_specs=[
            pl.BlockSpec((BR, D), lambda i: (i, 0)),
            pl.BlockSpec((NC, BR, D), lambda i: (0, i, 0)),
            pl.BlockSpec((D, D), lambda i: (0, 0)),
            pl.BlockSpec((D, D), lambda i: (0, 0)),
            pl.BlockSpec((1, D), lambda i: (0, 0)),
        ],
        out_specs=pl.BlockSpec((BR, D), lambda i: (i, 0)),
        out_shape=jax.ShapeDtypeStruct((N, D), jnp.float32),
    )(x, partials, W1, W2, b2.reshape(1, D))
    return out
